# baseline (speedup 1.0000x reference)
import jax
import jax.numpy as jnp
import numpy as np
from jax import lax
from jax.experimental import pallas as pl
from jax.experimental.pallas import tpu as pltpu
from jax.experimental.pallas import tpu_sc as plsc

_NC = 2
_NS = 16
_NW = _NC * _NS
_CHUNK = 128
_NBUF = 4


def _gather_body(table_hbm, idx_hbm, out_hbm, idx_v, bufs, stage, gsem, wsem):
    wid = lax.axis_index("s") * _NC + lax.axis_index("c")
    n_chunks = idx_v.shape[0]
    d = bufs.shape[2]
    row_bf = _CHUNK * d // 2
    base_chunk = wid * n_chunks
    pltpu.sync_copy(idx_hbm.at[pl.ds(base_chunk, n_chunks)], idx_v)

    def g_copy(j, b):
        return pltpu.make_async_copy(
            table_hbm.at[idx_v.at[j]], bufs.at[b], gsem.at[b])

    def w_copy(j, b):
        return pltpu.make_async_copy(
            stage.at[b],
            out_hbm.at[pl.ds((base_chunk + j) * row_bf, row_bf)],
            wsem.at[b])

    def convert(b):
        half = jnp.uint32(0x8000)
        hi_mask = jnp.uint32(0xFFFF0000)

        def row(i, carry):
            for g in range(d // 32):
                a = bufs[b, i, pl.ds(g * 32, 16)]
                v = bufs[b, i, pl.ds(g * 32 + 16, 16)]
                ua = lax.bitcast_convert_type(a, jnp.uint32)
                ub = lax.bitcast_convert_type(v, jnp.uint32)
                w = ((ua + half) >> 16) | ((ub + half) & hi_mask)
                stage[b, pl.ds(i * (d // 2) + g * 16, 16)] = lax.bitcast_convert_type(w, jnp.int32)
            return carry

        lax.fori_loop(0, _CHUNK, row, 0)

    n_outer = n_chunks // _NBUF

    def outer(g, carry):
        for b in range(_NBUF):
            j = g * _NBUF + b

            @pl.when(g > 0)
            def _():
                w_copy(j - _NBUF, b).wait()

            g_copy(j, b).start()
        for b in range(_NBUF):
            j = g * _NBUF + b
            g_copy(j, b).wait()
            convert(b)
            w_copy(j, b).start()
        return carry

    lax.fori_loop(0, n_outer, outer, 0)
    for b in range(_NBUF):
        j = (n_outer - 1) * _NBUF + b
        w_copy(j, b).wait()


def _matmul_body(x_ref, wt_ref, o_ref):
    bb, f, c = o_ref.shape
    h = f // 2
    d = wt_ref.shape[0]
    a = x_ref[...]
    o0 = jnp.dot(a[:, :d], wt_ref[...], preferred_element_type=jnp.float32)
    o1 = jnp.dot(a[:, d:], wt_ref[...], preferred_element_type=jnp.float32)
    r = jnp.concatenate([o0.reshape(bb, h, c), o1.reshape(bb, h, c)], axis=1)
    o_ref[...] = r


def kernel(data, emb_table, W):
    B, F = data.shape
    V, D = emb_table.shape
    C = W.shape[0]
    n = B * F
    assert n % (_NW * _CHUNK) == 0
    idx_perm = data.reshape(B, 2, F // 2).transpose(0, 2, 1).reshape(n)
    idx2d = idx_perm.reshape(n // _CHUNK, _CHUNK).astype(jnp.int32)
    chunks_per_w = (n // _CHUNK) // _NW

    gather = pl.kernel(
        _gather_body,
        out_type=jax.ShapeDtypeStruct((n * D // 2,), jnp.int32),
        mesh=plsc.VectorSubcoreMesh(core_axis_name="c", subcore_axis_name="s"),
        scratch_types=[
            pltpu.VMEM((chunks_per_w, _CHUNK), jnp.int32),
            pltpu.VMEM((_NBUF, _CHUNK, D), jnp.float32),
            pltpu.VMEM((_NBUF, _CHUNK * D // 2), jnp.int32),
            pltpu.SemaphoreType.DMA((_NBUF,)),
            pltpu.SemaphoreType.DMA((_NBUF,)),
        ],
        compiler_params=pltpu.CompilerParams(use_tc_tiling_on_sc=False),
    )
    x = gather(emb_table, idx2d)
    x2d = lax.bitcast_convert_type(x, jnp.bfloat16).reshape(n // 2, 2 * D)

    perm = np.array([32 * (p // 32) + (p % 32) // 2 + 16 * (p % 2)
                     for p in range(D)])
    wtp = jnp.asarray(W.T)[perm, :].astype(jnp.bfloat16)

    bb = 512
    out = pl.pallas_call(
        _matmul_body,
        out_shape=jax.ShapeDtypeStruct((B, F, C), jnp.float32),
        grid=(B // bb,),
        in_specs=[
            pl.BlockSpec(((F // 2) * bb, 2 * D), lambda i: (i, 0)),
            pl.BlockSpec((D, C), lambda i: (0, 0)),
        ],
        out_specs=pl.BlockSpec((bb, F, C), lambda i: (i, 0, 0)),
        compiler_params=pltpu.CompilerParams(
            dimension_semantics=("parallel",)),
    )(x2d, wtp)
    return out


# speedup vs baseline: 1.6767x; 1.6767x over previous
import jax
import jax.numpy as jnp
from jax import lax
from jax.experimental import pallas as pl
from jax.experimental.pallas import tpu as pltpu
from jax.experimental.pallas import tpu_sc as plsc

_NC = 2
_NS = 16
_NW = _NC * _NS
_CHUNK = 128
_NBUF = 4


def _gather_body(table_hbm, idx_hbm, out_hbm, idx_v, bufs, gsem, wsem):
    wid = lax.axis_index("s") * _NC + lax.axis_index("c")
    n_chunks = idx_v.shape[0]
    d = bufs.shape[2]
    base_chunk = wid * n_chunks
    pltpu.sync_copy(idx_hbm.at[pl.ds(base_chunk, n_chunks)], idx_v)

    def g_copy(j, b):
        return pltpu.make_async_copy(
            table_hbm.at[idx_v.at[j]], bufs.at[b], gsem.at[b])

    def w_copy(j, b):
        return pltpu.make_async_copy(
            bufs.at[b],
            out_hbm.at[pl.ds((base_chunk + j) * _CHUNK, _CHUNK), pl.ds(0, d)],
            wsem.at[b])

    n_outer = n_chunks // _NBUF

    def outer(g, carry):
        for b in range(_NBUF):
            j = g * _NBUF + b

            @pl.when(g > 0)
            def _():
                w_copy(j - _NBUF, b).wait()

            g_copy(j, b).start()
        for b in range(_NBUF):
            j = g * _NBUF + b
            g_copy(j, b).wait()
            w_copy(j, b).start()
        return carry

    lax.fori_loop(0, n_outer, outer, 0)
    for b in range(_NBUF):
        j = (n_outer - 1) * _NBUF + b
        w_copy(j, b).wait()


def _matmul_body(x_ref, wt_ref, o_ref):
    bb, f, c = o_ref.shape
    r = jnp.dot(x_ref[:, :wt_ref.shape[0]], wt_ref[...],
                preferred_element_type=jnp.float32)
    o_ref[...] = r.reshape(bb, f, c)


def kernel(data, emb_table, W):
    B, F = data.shape
    V, D = emb_table.shape
    C = W.shape[0]
    n = B * F
    assert n % (_NW * _CHUNK) == 0
    idx2d = data.reshape(n // _CHUNK, _CHUNK).astype(jnp.int32)
    chunks_per_w = (n // _CHUNK) // _NW

    gather = pl.kernel(
        _gather_body,
        out_type=jax.ShapeDtypeStruct((n, 2 * D), jnp.float32),
        mesh=plsc.VectorSubcoreMesh(core_axis_name="c", subcore_axis_name="s"),
        scratch_types=[
            pltpu.VMEM((chunks_per_w, _CHUNK), jnp.int32),
            pltpu.VMEM((_NBUF, _CHUNK, D), jnp.float32),
            pltpu.SemaphoreType.DMA((_NBUF,)),
            pltpu.SemaphoreType.DMA((_NBUF,)),
        ],
        compiler_params=pltpu.CompilerParams(use_tc_tiling_on_sc=False),
    )
    x = gather(emb_table, idx2d)

    wt = W.T
    bb = 512
    out = pl.pallas_call(
        _matmul_body,
        out_shape=jax.ShapeDtypeStruct((B, F, C), jnp.float32),
        grid=(B // bb,),
        in_specs=[
            pl.BlockSpec((F * bb, 2 * D), lambda i: (i, 0)),
            pl.BlockSpec((D, C), lambda i: (0, 0)),
        ],
        out_specs=pl.BlockSpec((bb, F, C), lambda i: (i, 0, 0)),
        compiler_params=pltpu.CompilerParams(
            dimension_semantics=("parallel",)),
    )(x, wt)
    return out


# speedup vs baseline: 1.6772x; 1.0003x over previous
import jax
import jax.numpy as jnp
from jax import lax
from jax.experimental import pallas as pl
from jax.experimental.pallas import tpu as pltpu
from jax.experimental.pallas import tpu_sc as plsc

_NC = 2
_NS = 16
_NW = _NC * _NS
_CHUNK = 128
_NBUF = 4


def _gather_body(table_hbm, idx_hbm, out_hbm, idx_v, bufs, gsem, wsem):
    wid = lax.axis_index("s") * _NC + lax.axis_index("c")
    n_chunks = idx_v.shape[0]
    d = bufs.shape[2]
    base_chunk = wid * n_chunks
    pltpu.sync_copy(idx_hbm.at[pl.ds(base_chunk, n_chunks)], idx_v)

    def g_copy(j, b):
        return pltpu.make_async_copy(
            table_hbm.at[idx_v.at[j]], bufs.at[b], gsem.at[b])

    def w_copy(j, b):
        return pltpu.make_async_copy(
            bufs.at[b],
            out_hbm.at[pl.ds((base_chunk + j) * _CHUNK, _CHUNK), pl.ds(0, d)],
            wsem.at[b])

    n_outer = n_chunks // _NBUF

    def outer(g, carry):
        for b in range(_NBUF):
            j = g * _NBUF + b

            @pl.when(g > 0)
            def _():
                w_copy(j - _NBUF, b).wait()

            g_copy(j, b).start()
        for b in range(_NBUF):
            j = g * _NBUF + b
            g_copy(j, b).wait()
            w_copy(j, b).start()
        return carry

    lax.fori_loop(0, n_outer, outer, 0)
    for b in range(_NBUF):
        j = (n_outer - 1) * _NBUF + b
        w_copy(j, b).wait()


def _matmul_body(x_hbm, wt_ref, o_ref, xbuf, sem):
    i = pl.program_id(0)
    nsteps = pl.num_programs(0)
    rows = xbuf.shape[1]

    def start(step, b):
        return pltpu.make_async_copy(
            x_hbm.at[pl.ds(step * rows, rows), :], xbuf.at[b], sem.at[b])

    @pl.when(i == 0)
    def _():
        start(0, 0).start()

    @pl.when(i + 1 < nsteps)
    def _():
        start(i + 1, (i + 1) % 2).start()

    start(i, i % 2).wait()
    bb, f, c = o_ref.shape
    r = jnp.dot(xbuf[i % 2, :, :wt_ref.shape[0]], wt_ref[...],
                preferred_element_type=jnp.float32)
    o_ref[...] = r.reshape(bb, f, c)


def kernel(data, emb_table, W):
    B, F = data.shape
    V, D = emb_table.shape
    C = W.shape[0]
    n = B * F
    assert n % (_NW * _CHUNK) == 0
    idx2d = data.reshape(n // _CHUNK, _CHUNK).astype(jnp.int32)
    chunks_per_w = (n // _CHUNK) // _NW

    gather = pl.kernel(
        _gather_body,
        out_type=jax.ShapeDtypeStruct((n, 2 * D), jnp.float32),
        mesh=plsc.VectorSubcoreMesh(core_axis_name="c", subcore_axis_name="s"),
        scratch_types=[
            pltpu.VMEM((chunks_per_w, _CHUNK), jnp.int32),
            pltpu.VMEM((_NBUF, _CHUNK, D), jnp.float32),
            pltpu.SemaphoreType.DMA((_NBUF,)),
            pltpu.SemaphoreType.DMA((_NBUF,)),
        ],
        compiler_params=pltpu.CompilerParams(use_tc_tiling_on_sc=False),
    )
    x = gather(emb_table, idx2d)

    wt = W.T
    bb = 512
    out = pl.pallas_call(
        _matmul_body,
        out_shape=jax.ShapeDtypeStruct((B, F, C), jnp.float32),
        grid=(B // bb,),
        in_specs=[
            pl.BlockSpec(memory_space=pl.ANY),
            pl.BlockSpec((D, C), lambda i: (0, 0)),
        ],
        out_specs=pl.BlockSpec((bb, F, C), lambda i: (i, 0, 0)),
        scratch_shapes=[
            pltpu.VMEM((2, F * bb, 2 * D), jnp.float32),
            pltpu.SemaphoreType.DMA((2,)),
        ],
        compiler_params=pltpu.CompilerParams(
            dimension_semantics=("arbitrary",)),
    )(x, wt)
    return out


# speedup vs baseline: 1.6802x; 1.0017x over previous
import jax
import jax.numpy as jnp
from jax import lax
from jax.experimental import pallas as pl
from jax.experimental.pallas import tpu as pltpu
from jax.experimental.pallas import tpu_sc as plsc

_NC = 2
_NS = 16
_NW = _NC * _NS
_CHUNK = 128
_NBUF = 4


def _gather_body(table_hbm, idx_hbm, out_hbm, idx_v, bufs, gsem, wsem):
    wid = lax.axis_index("s") * _NC + lax.axis_index("c")
    n_chunks = idx_v.shape[0]
    d = bufs.shape[2]
    base_chunk = wid * n_chunks
    pltpu.sync_copy(idx_hbm.at[pl.ds(base_chunk, n_chunks)], idx_v)

    def g_copy(j, b):
        return pltpu.make_async_copy(
            table_hbm.at[idx_v.at[j]], bufs.at[b], gsem.at[b])

    def w_copy(j, b):
        return pltpu.make_async_copy(
            bufs.at[b],
            out_hbm.at[pl.ds((base_chunk + j) * _CHUNK, _CHUNK), pl.ds(0, d)],
            wsem.at[b])

    n_outer = n_chunks // _NBUF

    def outer(g, carry):
        for b in range(_NBUF):
            j = g * _NBUF + b

            @pl.when(g > 0)
            def _():
                w_copy(j - _NBUF, b).wait()

            g_copy(j, b).start()
        for b in range(_NBUF):
            j = g * _NBUF + b
            g_copy(j, b).wait()
            w_copy(j, b).start()
        return carry

    lax.fori_loop(0, n_outer, outer, 0)
    for b in range(_NBUF):
        j = (n_outer - 1) * _NBUF + b
        w_copy(j, b).wait()


def _matmul_body(x_ref, wt_ref, o_ref):
    bb, f, c = o_ref.shape
    r = jnp.dot(x_ref[:, :wt_ref.shape[0]], wt_ref[...],
                preferred_element_type=jnp.float32)
    o_ref[...] = r.reshape(bb, f, c)


def kernel(data, emb_table, W):
    B, F = data.shape
    V, D = emb_table.shape
    C = W.shape[0]
    n = B * F
    assert n % (_NW * _CHUNK) == 0
    idx2d = data.reshape(n // _CHUNK, _CHUNK).astype(jnp.int32)
    chunks_per_w = (n // _CHUNK) // _NW

    gather = pl.kernel(
        _gather_body,
        out_type=jax.ShapeDtypeStruct((n, 2 * D), jnp.float32),
        mesh=plsc.VectorSubcoreMesh(core_axis_name="c", subcore_axis_name="s"),
        scratch_types=[
            pltpu.VMEM((chunks_per_w, _CHUNK), jnp.int32),
            pltpu.VMEM((_NBUF, _CHUNK, D), jnp.float32),
            pltpu.SemaphoreType.DMA((_NBUF,)),
            pltpu.SemaphoreType.DMA((_NBUF,)),
        ],
        compiler_params=pltpu.CompilerParams(use_tc_tiling_on_sc=False),
    )
    x = gather(emb_table, idx2d)

    wt = W.T
    bb = 1024
    out = pl.pallas_call(
        _matmul_body,
        out_shape=jax.ShapeDtypeStruct((B, F, C), jnp.float32),
        grid=(B // bb,),
        in_specs=[
            pl.BlockSpec((F * bb, 2 * D), lambda i: (i, 0)),
            pl.BlockSpec((D, C), lambda i: (0, 0)),
        ],
        out_specs=pl.BlockSpec((bb, F, C), lambda i: (i, 0, 0)),
        compiler_params=pltpu.CompilerParams(
            dimension_semantics=("parallel",)),
    )(x, wt)
    return out
